# baseline (speedup 1.0000x reference)
import functools

import jax
import jax.numpy as jnp
from jax.experimental import pallas as pl
from jax.experimental.pallas import tpu as pltpu


def _phase_kernel(inp_ref, tgt_ref, table_ref, out_ref, *, num_labels,
                  embed_dim, bs, hs, ws):
    c = pl.program_id(0)
    ce = jax.lax.rem(c, embed_dim)
    idx = jnp.where(c < embed_dim, inp_ref[...], tgt_ref[...])
    row = jnp.zeros((1, bs), jnp.float32)
    for lbl in range(num_labels):
        row = jnp.where(idx == lbl, table_ref[lbl, ce], row)
    out_ref[0] = jnp.broadcast_to(row[:, None, :], (out_ref.shape[1], ws, bs))


def kernel(table, inp_idx, tgt_idx, B, H, W):
    Bs = inp_idx.shape[0]
    num_labels, embed_dim = table.shape
    Hs, Ws = 64, 64
    C = 2 * embed_dim

    out_chwb = pl.pallas_call(
        functools.partial(_phase_kernel, num_labels=num_labels,
                          embed_dim=embed_dim, bs=Bs, hs=Hs, ws=Ws),
        grid=(C, 2),
        in_specs=[
            pl.BlockSpec((1, Bs), lambda c, h: (0, 0)),
            pl.BlockSpec((1, Bs), lambda c, h: (0, 0)),
            pl.BlockSpec(memory_space=pltpu.SMEM),
        ],
        out_specs=pl.BlockSpec((1, Hs // 2, Ws, Bs),
                               lambda c, h: (c, h, 0, 0)),
        out_shape=jax.ShapeDtypeStruct((C, Hs, Ws, Bs), jnp.float32),
    )(inp_idx.reshape(1, Bs), tgt_idx.reshape(1, Bs), table)
    return jnp.transpose(out_chwb, (3, 0, 1, 2))


# speedup vs baseline: 1.1853x; 1.1853x over previous
import functools

import jax
import jax.numpy as jnp
from jax.experimental import pallas as pl
from jax.experimental.pallas import tpu as pltpu


def _phase_kernel(inp_ref, tgt_ref, table_ref, out_ref, *, num_labels,
                  embed_dim, bs, hs, ws):
    c = pl.program_id(0)
    ce = jax.lax.rem(c, embed_dim)
    idx = jnp.where(c < embed_dim, inp_ref[...], tgt_ref[...])
    row = jnp.zeros((1, bs), jnp.float32)
    for lbl in range(num_labels):
        row = jnp.where(idx == lbl, table_ref[lbl, ce], row)
    out_ref[0] = jnp.broadcast_to(row[:, None, :], (hs, ws, bs))


def kernel(table, inp_idx, tgt_idx, B, H, W):
    Bs = inp_idx.shape[0]
    num_labels, embed_dim = table.shape
    Hs, Ws = 64, 64
    C = 2 * embed_dim

    out_chwb = pl.pallas_call(
        functools.partial(_phase_kernel, num_labels=num_labels,
                          embed_dim=embed_dim, bs=Bs, hs=Hs, ws=Ws),
        grid=(C,),
        in_specs=[
            pl.BlockSpec((1, Bs), lambda c: (0, 0)),
            pl.BlockSpec((1, Bs), lambda c: (0, 0)),
            pl.BlockSpec(memory_space=pltpu.SMEM),
        ],
        out_specs=pl.BlockSpec((1, Hs, Ws, Bs), lambda c: (c, 0, 0, 0)),
        out_shape=jax.ShapeDtypeStruct((C, Hs, Ws, Bs), jnp.float32),
    )(inp_idx.reshape(1, Bs), tgt_idx.reshape(1, Bs), table)
    return jnp.transpose(out_chwb, (3, 0, 1, 2))
